# baseline (speedup 1.0000x reference)
import functools

import jax
import jax.numpy as jnp
from jax import lax
from jax.experimental import pallas as pl
from jax.experimental.pallas import tpu as pltpu
from jax.experimental.pallas import tpu_sc as plsc

BATCH = 4096
HIST = 50
EMBED_DIM = 64

NUM_CORES = 2
NUM_SUBCORES = 16
NUM_WORKERS = NUM_CORES * NUM_SUBCORES
BATCH_PER_WORKER = BATCH // NUM_WORKERS
GROUP_B = 16
N_GROUPS = BATCH_PER_WORKER // GROUP_B
GROUP_IDX = GROUP_B * HIST


def _build():
    mesh = plsc.VectorSubcoreMesh(core_axis_name="c", subcore_axis_name="s")

    @functools.partial(
        pl.kernel,
        mesh=mesh,
        out_type=jax.ShapeDtypeStruct((HIST, 8, NUM_WORKERS, 8, 128),
                                      jnp.float32),
        scratch_types=[
            pltpu.VMEM((N_GROUPS, GROUP_IDX), jnp.int32),
            pltpu.VMEM((2, GROUP_IDX, EMBED_DIM), jnp.float32),
            pltpu.VMEM((2, 8, 8, GROUP_B), jnp.float32),
            pltpu.SemaphoreType.DMA((2,)),
            pltpu.SemaphoreType.DMA((2,)),
        ],
        compiler_params=pltpu.CompilerParams(use_tc_tiling_on_sc=False,
                                             needs_layout_passes=False),
    )
    def gather_kernel(idx_hbm, table_hbm, out_hbm, idx_v, rows_v, tbuf, gsem,
                      wsem):
        wid = lax.axis_index("s") * NUM_CORES + lax.axis_index("c")

        pltpu.sync_copy(idx_hbm.at[wid], idx_v)

        def idx_at(g):
            return idx_v.at[g]

        lanes = lax.iota(jnp.int32, 16)
        row_base = lanes * HIST

        pltpu.async_copy(table_hbm.at[idx_at(0)], rows_v.at[0], gsem.at[0])

        def transpose_h(rows, h, p):
            rows_of_h = row_base + h
            for d in range(EMBED_DIM):
                v = plsc.load_gather(rows, [rows_of_h, jnp.full((16,), d,
                                                                jnp.int32)])
                tbuf[p, d // 8, d % 8] = v

        for g in range(N_GROUPS):
            b = g % 2
            pltpu.make_async_copy(table_hbm.at[idx_at(g)], rows_v.at[b],
                                  gsem.at[b]).wait()
            if g + 1 < N_GROUPS:
                pltpu.async_copy(table_hbm.at[idx_at(g + 1)],
                                 rows_v.at[1 - b], gsem.at[1 - b])

            def pair(hh, _, b=b, g=g):
                for par in range(2):
                    h = hh * 2 + par

                    def drain(par=par):
                        pltpu.make_async_copy(
                            tbuf.at[par],
                            out_hbm.at[0, pl.ds(0, 8), wid, pl.ds(0, 8),
                                       pl.ds(0, GROUP_B)],
                            wsem.at[par]).wait()

                    if g == 0:
                        pl.when(hh > 0)(drain)
                    else:
                        drain()
                    transpose_h(rows_v.at[b], h, par)
                    pltpu.async_copy(
                        tbuf.at[par],
                        out_hbm.at[h, pl.ds(0, 8), wid, pl.ds(0, 8),
                                   pl.ds(g * GROUP_B, GROUP_B)],
                        wsem.at[par])
                return ()

            lax.fori_loop(0, HIST // 2, pair, (), unroll=False)

        for par in range(2):
            pltpu.make_async_copy(
                tbuf.at[par],
                out_hbm.at[0, pl.ds(0, 8), wid, pl.ds(0, 8),
                           pl.ds(0, GROUP_B)],
                wsem.at[par]).wait()

    return gather_kernel


_gather = _build()


@jax.jit
def kernel(x, table):
    idx3d = x.reshape(NUM_WORKERS, N_GROUPS, GROUP_IDX)
    out5d = _gather(idx3d, table)
    return out5d.transpose(2, 4, 0, 1, 3).reshape(BATCH, HIST, EMBED_DIM)


# speedup vs baseline: 2.4173x; 2.4173x over previous
import functools

import jax
import jax.numpy as jnp
from jax import lax
from jax.experimental import pallas as pl
from jax.experimental.pallas import tpu as pltpu
from jax.experimental.pallas import tpu_sc as plsc

BATCH = 4096
HIST = 50
EMBED_DIM = 64
HIST_PAD = 56

NUM_CORES = 2
NUM_SUBCORES = 16
NUM_WORKERS = NUM_CORES * NUM_SUBCORES
BATCH_PER_WORKER = BATCH // NUM_WORKERS
IDX_ROWS_PER_WORKER = BATCH_PER_WORKER * HIST // 128
BATCHES_PER_CHUNK = 8
CHUNK = BATCHES_PER_CHUNK * HIST
N_CHUNKS = BATCH_PER_WORKER // BATCHES_PER_CHUNK
NBUF = 4


def _build():
    mesh = plsc.VectorSubcoreMesh(core_axis_name="c", subcore_axis_name="s")

    @functools.partial(
        pl.kernel,
        mesh=mesh,
        out_type=jax.ShapeDtypeStruct((BATCH, HIST_PAD, 128), jnp.float32),
        scratch_types=[
            pltpu.VMEM((1, BATCH_PER_WORKER * HIST), jnp.int32),
            pltpu.VMEM((NBUF, CHUNK, EMBED_DIM), jnp.float32),
            pltpu.SemaphoreType.DMA((NBUF,)),
        ],
        compiler_params=pltpu.CompilerParams(use_tc_tiling_on_sc=False),
    )
    def gather_kernel(idx_hbm, table_hbm, out_hbm, idx_v, rows_v, sems):
        wid = lax.axis_index("s") * NUM_CORES + lax.axis_index("c")
        base = wid * BATCH_PER_WORKER

        pltpu.sync_copy(idx_hbm.at[pl.ds(wid, 1)], idx_v)

        def idx_at(c):
            return idx_v.at[0, pl.ds(c * CHUNK, CHUNK)]

        for b in range(NBUF):
            pltpu.async_copy(table_hbm.at[idx_at(b)], rows_v.at[b],
                             sems.at[b])

        def group(g, _):
            for b in range(NBUF):
                c = g * NBUF + b
                pltpu.make_async_copy(table_hbm.at[idx_at(b)],
                                      rows_v.at[b], sems.at[b]).wait()
                for j in range(BATCHES_PER_CHUNK):
                    pltpu.sync_copy(
                        rows_v.at[b].at[pl.ds(j * HIST, HIST)],
                        out_hbm.at[base + c * BATCHES_PER_CHUNK + j,
                                   pl.ds(0, HIST), pl.ds(0, EMBED_DIM)])
                nxt = c + NBUF

                @pl.when(nxt < N_CHUNKS)
                def _():
                    pltpu.async_copy(table_hbm.at[idx_at(nxt)],
                                     rows_v.at[b], sems.at[b])

            return ()

        lax.fori_loop(0, N_CHUNKS // NBUF, group, (), unroll=False)

    return gather_kernel


_gather = _build()


@jax.jit
def kernel(x, table):
    idx2d = x.reshape(NUM_WORKERS, BATCH_PER_WORKER * HIST)
    out = _gather(idx2d, table)
    return out[:, :HIST, :EMBED_DIM]
